# baseline (speedup 1.0000x reference)
import dataclasses
import functools

import jax
import jax.numpy as jnp
from jax import lax
from jax.experimental import pallas as pl
from jax.experimental.pallas import tpu as pltpu
from jax.experimental.pallas import tpu_sc as plsc

N = 10000
E = 320000
FIN = 128
FQK = 128
FV = 128
H = 8
FH = 16
NTILES = 32
NP = 10240
ND = NP // 8
RPT = NP // 16
DPT = ND // 16
ZB = 16

EA = 128
EC = 64

EPT0 = 4864
EPT1 = 5136
OFF1 = EPT0 * NTILES


def _compiler_params():
    cp = pltpu.CompilerParams()
    if "needs_layout_passes" in pltpu.CompilerParams.__dataclass_fields__:
        cp = dataclasses.replace(cp, needs_layout_passes=False)
    return cp


def _mesh():
    return plsc.VectorSubcoreMesh(core_axis_name="c", subcore_axis_name="s")


def _proj_body(x_ref, w_ref, q_ref, kv_ref):
    p = lax.dot_general(x_ref[...], w_ref[...], (((1,), (1,)), ((), ())),
                        preferred_element_type=jnp.float32)
    q_ref[...] = p[:, :FQK] * (FH ** -0.5)
    kv_ref[...] = p[:, FQK:]


def _project(x, W):
    blk = 1000
    return pl.pallas_call(
        _proj_body,
        grid=(N // blk,),
        in_specs=[
            pl.BlockSpec((blk, FIN), lambda i: (i, 0)),
            pl.BlockSpec((2 * FQK + FV, FIN), lambda i: (0, 0)),
        ],
        out_specs=[
            pl.BlockSpec((blk, FQK), lambda i: (i, 0)),
            pl.BlockSpec((blk, FQK + FV), lambda i: (i, 0)),
        ],
        out_shape=[
            jax.ShapeDtypeStruct((N, FQK), jnp.float32),
            jax.ShapeDtypeStruct((N, FQK + FV), jnp.float32),
        ],
    )(x, W)


def _make_gather_body(off, ept):
    sa = ept // EA
    ta = ept - sa * EA

    def body(q_hbm, kv_hbm, src_hbm, dst_hbm, qs_hbm, kvs_hbm,
             s0, d0, q0, k0, s1, d1, q1, k1, st, dt,
             i0, i1, g0, g1, w0, w1):
        cid = lax.axis_index("c")
        sid = lax.axis_index("s")
        lb = (cid * 16 + sid) * ept
        gb = off + lb
        S = (s0, s1)
        D = (d0, d1)
        Q = (q0, q1)
        K = (k0, k1)
        SI = (i0, i1)
        SG = (g0, g1)
        SW = (w0, w1)

        def issue_idx(b, step):
            base = gb + step * EA
            pltpu.async_copy(src_hbm.at[pl.ds(base, EA)], S[b], SI[b])
            pltpu.async_copy(dst_hbm.at[pl.ds(base, EA)], D[b], SI[b])

        def wait_idx(b):
            pltpu.make_async_copy(src_hbm.at[pl.ds(0, EA)], S[b], SI[b]).wait()
            pltpu.make_async_copy(dst_hbm.at[pl.ds(0, EA)], D[b], SI[b]).wait()

        def issue_gather(b):
            pltpu.async_copy(q_hbm.at[S[b]], Q[b], SG[b])
            pltpu.async_copy(kv_hbm.at[D[b]], K[b], SG[b])

        def wait_gather(b):
            pltpu.make_async_copy(q_hbm.at[S[b]], Q[b], SG[b]).wait()
            pltpu.make_async_copy(kv_hbm.at[D[b]], K[b], SG[b]).wait()

        def issue_write(b, step):
            base = lb + step * EA
            pltpu.async_copy(Q[b], qs_hbm.at[pl.ds(base, EA)], SW[b])
            pltpu.async_copy(K[b], kvs_hbm.at[pl.ds(base, EA)], SW[b])

        def wait_write(b):
            pltpu.make_async_copy(Q[b], qs_hbm.at[pl.ds(0, EA)], SW[b]).wait()
            pltpu.make_async_copy(K[b], kvs_hbm.at[pl.ds(0, EA)], SW[b]).wait()

        issue_idx(0, 0)
        issue_idx(1, 1)
        wait_idx(0)
        issue_gather(0)

        @pl.loop(0, sa // 2)
        def _(it):
            for b in (0, 1):
                i = it * 2 + b
                nb = 1 - b

                @pl.when(i + 1 < sa)
                def _():
                    wait_idx(nb)

                @pl.when(jnp.logical_and(i + 1 < sa, i >= 1))
                def _():
                    wait_write(nb)

                @pl.when(i + 1 < sa)
                def _():
                    issue_gather(nb)

                wait_gather(b)
                issue_write(b, i)

                @pl.when(i + 2 < sa)
                def _():
                    issue_idx(b, i + 2)

        wait_write(0)
        wait_write(1)

        if ta:
            pltpu.sync_copy(src_hbm.at[pl.ds(gb + sa * EA, ta)], st)
            pltpu.sync_copy(dst_hbm.at[pl.ds(gb + sa * EA, ta)], dt)
            pltpu.sync_copy(q_hbm.at[st], q0.at[pl.ds(0, ta)])
            pltpu.sync_copy(kv_hbm.at[dt], k0.at[pl.ds(0, ta)])
            pltpu.sync_copy(q0.at[pl.ds(0, ta)],
                            qs_hbm.at[pl.ds(lb + sa * EA, ta)])
            pltpu.sync_copy(k0.at[pl.ds(0, ta)],
                            kvs_hbm.at[pl.ds(lb + sa * EA, ta)])

    return body


def _sc_gather(q_tbl, kv_tbl, src, dst, off, ept):
    ec = ept * NTILES
    ta = max(ept - (ept // EA) * EA, 8)
    dbuf = [
        pltpu.VMEM((EA,), jnp.int32),
        pltpu.VMEM((EA,), jnp.int32),
        pltpu.VMEM((EA, FQK), jnp.float32),
        pltpu.VMEM((EA, FQK + FV), jnp.float32),
    ]
    fn = pl.kernel(
        _make_gather_body(off, ept),
        compiler_params=_compiler_params(),
        out_type=[
            jax.ShapeDtypeStruct((ec, FQK), jnp.float32),
            jax.ShapeDtypeStruct((ec, FQK + FV), jnp.float32),
        ],
        mesh=_mesh(),
        scratch_types=dbuf + dbuf + [
            pltpu.VMEM((ta,), jnp.int32),
            pltpu.VMEM((ta,), jnp.int32),
        ] + [pltpu.SemaphoreType.DMA] * 6,
    )
    return fn(q_tbl, kv_tbl, src, dst)


def _edge_body(src_ref, qs_ref, kvs_ref, wv_ref, dn_ref):
    blk = qs_ref.shape[0]
    qs = qs_ref[...]
    ks = kvs_ref[:, :FQK]
    vs = kvs_ref[:, FQK:]
    prod = qs * ks
    ch = lax.broadcasted_iota(jnp.int32, (FQK, H), 0) // FH
    hh = lax.broadcasted_iota(jnp.int32, (FQK, H), 1)
    sum16 = (ch == hh).astype(jnp.float32)
    hr = lax.broadcasted_iota(jnp.int32, (H, FV), 0)
    hc = lax.broadcasted_iota(jnp.int32, (H, FV), 1)
    expand = ((hc // FH) == hr).astype(jnp.float32)
    tile8 = ((hc & 15) == hr).astype(jnp.float32)

    aw = lax.dot_general(prod, sum16, (((1,), (0,)), ((), ())),
                         preferred_element_type=jnp.float32)
    w = jnp.exp(aw)
    wrep = lax.dot_general(w, expand, (((1,), (0,)), ((), ())),
                           preferred_element_type=jnp.float32)
    wv_ref[...] = wrep * vs
    wtile = lax.dot_general(w, tile8, (((1,), (0,)), ((), ())),
                            preferred_element_type=jnp.float32)
    grp = jnp.broadcast_to(src_ref[...] & 7, (blk, FV))
    lane16 = lax.broadcasted_iota(jnp.int32, (blk, FV), 1) // FH
    dn_ref[...] = jnp.where(lane16 == grp, wtile, 0.0)


def _edge_compute(src2, qs, kvs):
    blk = 512
    ec = qs.shape[0]
    return pl.pallas_call(
        _edge_body,
        grid=(ec // blk,),
        in_specs=[
            pl.BlockSpec((blk, 1), lambda i: (i, 0)),
            pl.BlockSpec((blk, FQK), lambda i: (i, 0)),
            pl.BlockSpec((blk, FQK + FV), lambda i: (i, 0)),
        ],
        out_specs=[
            pl.BlockSpec((blk, FV), lambda i: (i, 0)),
            pl.BlockSpec((blk, 128), lambda i: (i, 0)),
        ],
        out_shape=[
            jax.ShapeDtypeStruct((ec, FV), jnp.float32),
            jax.ShapeDtypeStruct((ec, 128), jnp.float32),
        ],
    )(src2, qs, kvs)


def _make_scatter_body(off, ept):
    sc = ept // EC
    tc = ept - sc * EC

    def body(wv_hbm, dn_hbm, src_hbm, num_hbm, den_hbm,
             s0, w0, n0, s1, w1, n1, st, didx, dt_, zv,
             acc_n, acc_d, i0, i1, l0, l1, sz):
        cid = lax.axis_index("c")
        sid = lax.axis_index("s")
        lb = (cid * 16 + sid) * ept
        gb = off + lb
        zero16 = jnp.zeros((16,), jnp.float32)
        S = (s0, s1)
        Wb = (w0, w1)
        Nb = (n0, n1)
        SI = (i0, i1)
        SL = (l0, l1)

        @pl.loop(0, ZB)
        def _(i):
            for j in range(FV // 16):
                zv[i, pl.ds(16 * j, 16)] = zero16

        ztargets = [acc_n.at[pl.ds(sid * RPT + r * ZB, ZB)]
                    for r in range(RPT // ZB)]
        ztargets += [acc_d.at[pl.ds(sid * DPT + r * ZB, ZB)]
                     for r in range(DPT // ZB)]
        for wave in range(0, len(ztargets), 15):
            hs = [pltpu.async_copy(zv, t, sz)
                  for t in ztargets[wave:wave + 15]]
            for hh in hs:
                hh.wait()
        plsc.subcore_barrier()

        def issue_load(b, step):
            pltpu.async_copy(src_hbm.at[pl.ds(gb + step * EC, EC)],
                             S[b], SI[b])
            pltpu.async_copy(wv_hbm.at[pl.ds(lb + step * EC, EC)],
                             Wb[b], SL[b])
            pltpu.async_copy(dn_hbm.at[pl.ds(lb + step * EC, EC)],
                             Nb[b], SL[b])

        def wait_load(b):
            pltpu.make_async_copy(src_hbm.at[pl.ds(0, EC)],
                                  S[b], SI[b]).wait()
            pltpu.make_async_copy(wv_hbm.at[pl.ds(0, EC)],
                                  Wb[b], SL[b]).wait()
            pltpu.make_async_copy(dn_hbm.at[pl.ds(0, EC)],
                                  Nb[b], SL[b]).wait()

        issue_load(0, 0)
        issue_load(1, 1)

        @pl.loop(0, sc // 2)
        def _(it):
            for b in (0, 1):
                i = it * 2 + b
                wait_load(b)

                @pl.loop(0, EC // 16)
                def _(c):
                    didx[pl.ds(c * 16, 16)] = lax.shift_right_logical(
                        S[b][pl.ds(c * 16, 16)], 3)

                pltpu.sync_copy(Wb[b], acc_n.at[S[b]], add=True)
                pltpu.sync_copy(Nb[b], acc_d.at[didx], add=True)

                @pl.when(i + 2 < sc)
                def _():
                    issue_load(b, i + 2)

        if tc:
            pltpu.sync_copy(src_hbm.at[pl.ds(gb + sc * EC, tc)], st)
            pltpu.sync_copy(wv_hbm.at[pl.ds(lb + sc * EC, tc)],
                            w0.at[pl.ds(0, tc)])
            pltpu.sync_copy(dn_hbm.at[pl.ds(lb + sc * EC, tc)],
                            n0.at[pl.ds(0, tc)])
            dt_[pl.ds(0, 16)] = lax.shift_right_logical(st[pl.ds(0, 16)], 3)
            pltpu.sync_copy(w0.at[pl.ds(0, tc)], acc_n.at[st], add=True)
            pltpu.sync_copy(n0.at[pl.ds(0, tc)], acc_d.at[dt_], add=True)

        plsc.subcore_barrier()
        for r in range(RPT // EC):
            pltpu.sync_copy(acc_n.at[pl.ds(sid * RPT + r * EC, EC)], w0)
            pltpu.sync_copy(w0,
                            num_hbm.at[cid, pl.ds(sid * RPT + r * EC, EC)])
        for r in range(DPT // 16):
            pltpu.sync_copy(acc_d.at[pl.ds(sid * DPT + r * 16, 16)],
                            w0.at[pl.ds(0, 16)])
            pltpu.sync_copy(w0.at[pl.ds(0, 16)],
                            den_hbm.at[cid, pl.ds(sid * DPT + r * 16, 16)])

    return body


def _sc_scatter(wv, dn, src, off, ept):
    tc = max(ept - (ept // EC) * EC, 8)
    dbuf = [
        pltpu.VMEM((EC,), jnp.int32),
        pltpu.VMEM((EC, FV), jnp.float32),
        pltpu.VMEM((EC, 128), jnp.float32),
    ]
    fn = pl.kernel(
        _make_scatter_body(off, ept),
        compiler_params=_compiler_params(),
        out_type=[
            jax.ShapeDtypeStruct((2, NP, FV), jnp.float32),
            jax.ShapeDtypeStruct((2, ND, 128), jnp.float32),
        ],
        mesh=_mesh(),
        scratch_types=dbuf + dbuf + [
            pltpu.VMEM((tc,), jnp.int32),
            pltpu.VMEM((EC,), jnp.int32),
            pltpu.VMEM((tc,), jnp.int32),
            pltpu.VMEM((ZB, 128), jnp.float32),
            pltpu.VMEM_SHARED((NP, FV), jnp.float32),
            pltpu.VMEM_SHARED((ND, 128), jnp.float32),
        ] + [pltpu.SemaphoreType.DMA] * 5,
    )
    return fn(wv, dn, src)


def _comb_body(num_a, den_a, num_b, den_b, o_ref):
    num = num_a[0] + num_a[1] + num_b[0] + num_b[1]
    den16 = den_a[0] + den_a[1] + den_b[0] + den_b[1]
    col = lax.broadcasted_iota(jnp.int32, (16, FV), 1) // FH
    row = lax.broadcasted_iota(jnp.int32, (16, FV), 0)
    ex = (col == row).astype(jnp.float32)
    rep = lax.dot_general(den16, ex, (((1,), (0,)), ((), ())),
                          preferred_element_type=jnp.float32)
    o_ref[...] = jnp.where(rep > 0, num / rep, 0.0)


def _combine(num_a, den_a, num_b, den_b):
    blk = 1000
    nspec = pl.BlockSpec((2, blk, FV), lambda i: (0, i, 0))
    dspec = pl.BlockSpec((2, blk, 16), lambda i: (0, i, 0))
    return pl.pallas_call(
        _comb_body,
        grid=(N // blk,),
        in_specs=[nspec, dspec, nspec, dspec],
        out_specs=pl.BlockSpec((blk, FV), lambda i: (i, 0)),
        out_shape=jax.ShapeDtypeStruct((N, FV), jnp.float32),
    )(num_a, den_a, num_b, den_b)


def kernel(x, batch, ei, W):
    del batch
    src = ei[0]
    dst = ei[1]
    src2 = src.reshape(E, 1)
    q_tbl, kv_tbl = _project(x, W)

    qs0, kvs0 = _sc_gather(q_tbl, kv_tbl, src, dst, 0, EPT0)
    qs1, kvs1 = _sc_gather(q_tbl, kv_tbl, src, dst, OFF1, EPT1)
    wv0, dn0 = _edge_compute(src2[:OFF1], qs0, kvs0)
    wv1, dn1 = _edge_compute(src2[OFF1:], qs1, kvs1)
    num0, den0 = _sc_scatter(wv0, dn0, src, 0, EPT0)
    num1, den1 = _sc_scatter(wv1, dn1, src, OFF1, EPT1)
    return _combine(num0, den0.reshape(2, NP, 16),
                    num1, den1.reshape(2, NP, 16))


# speedup vs baseline: 1.1761x; 1.1761x over previous
import dataclasses
import functools

import jax
import jax.numpy as jnp
from jax import lax
from jax.experimental import pallas as pl
from jax.experimental.pallas import tpu as pltpu
from jax.experimental.pallas import tpu_sc as plsc

N = 10000
E = 320000
FIN = 128
FQK = 128
FV = 128
H = 8
FH = 16
NTILES = 32
EPT = E // NTILES
NP = 10240
ND = NP // 8
RPT = NP // 16
DPT = ND // 16
ZB = 16

EA = 80
SA = EPT // EA
NBUF = 4

EC = 64
SC = EPT // EC
TC = EPT - SC * EC


def _compiler_params():
    cp = pltpu.CompilerParams()
    if "needs_layout_passes" in pltpu.CompilerParams.__dataclass_fields__:
        cp = dataclasses.replace(cp, needs_layout_passes=False)
    return cp


def _mesh():
    return plsc.VectorSubcoreMesh(core_axis_name="c", subcore_axis_name="s")


def _proj_body(x_ref, w_ref, q_ref, kv_ref):
    p = lax.dot_general(x_ref[...], w_ref[...], (((1,), (1,)), ((), ())),
                        preferred_element_type=jnp.float32)
    q_ref[...] = p[:, :FQK] * (FH ** -0.5)
    kv_ref[...] = p[:, FQK:]


def _project(x, W):
    blk = 1000
    return pl.pallas_call(
        _proj_body,
        grid=(N // blk,),
        in_specs=[
            pl.BlockSpec((blk, FIN), lambda i: (i, 0)),
            pl.BlockSpec((2 * FQK + FV, FIN), lambda i: (0, 0)),
        ],
        out_specs=[
            pl.BlockSpec((blk, FQK), lambda i: (i, 0)),
            pl.BlockSpec((blk, FQK + FV), lambda i: (i, 0)),
        ],
        out_shape=[
            jax.ShapeDtypeStruct((N, FQK), jnp.float32),
            jax.ShapeDtypeStruct((N, FQK + FV), jnp.float32),
        ],
    )(x, W)


def _sc_gather_body(q_hbm, kv_hbm, src_hbm, dst_hbm, qs_hbm, kvs_hbm,
                    *scratch):
    S = scratch[0:NBUF]
    D = scratch[NBUF:2 * NBUF]
    Q = scratch[2 * NBUF:3 * NBUF]
    K = scratch[3 * NBUF:4 * NBUF]
    SI = scratch[4 * NBUF:5 * NBUF]
    SG = scratch[5 * NBUF:6 * NBUF]
    SW = scratch[6 * NBUF:7 * NBUF]
    cid = lax.axis_index("c")
    sid = lax.axis_index("s")
    tb = (cid * 16 + sid) * EPT

    def issue_idx(u, step):
        base = tb + step * EA
        pltpu.async_copy(src_hbm.at[pl.ds(base, EA)], S[u], SI[u])
        pltpu.async_copy(dst_hbm.at[pl.ds(base, EA)], D[u], SI[u])

    def wait_idx(u):
        pltpu.make_async_copy(src_hbm.at[pl.ds(0, EA)], S[u], SI[u]).wait()
        pltpu.make_async_copy(dst_hbm.at[pl.ds(0, EA)], D[u], SI[u]).wait()

    def issue_gather(u):
        pltpu.async_copy(q_hbm.at[S[u]], Q[u], SG[u])
        pltpu.async_copy(kv_hbm.at[D[u]], K[u], SG[u])

    def wait_gather(u):
        pltpu.make_async_copy(q_hbm.at[S[u]], Q[u], SG[u]).wait()
        pltpu.make_async_copy(kv_hbm.at[D[u]], K[u], SG[u]).wait()

    def issue_write(u, step):
        base = tb + step * EA
        pltpu.async_copy(Q[u], qs_hbm.at[pl.ds(base, EA)], SW[u])
        pltpu.async_copy(K[u], kvs_hbm.at[pl.ds(base, EA)], SW[u])

    def wait_write(u):
        pltpu.make_async_copy(Q[u], qs_hbm.at[pl.ds(0, EA)], SW[u]).wait()
        pltpu.make_async_copy(K[u], kvs_hbm.at[pl.ds(0, EA)], SW[u]).wait()

    issue_idx(0, 0)
    issue_idx(1, 1)
    issue_idx(2, 2)
    wait_idx(0)
    issue_gather(0)

    def step_fn(u, i):
        nu = (u + 1) % NBUF
        pu = (u + NBUF - 1) % NBUF

        @pl.when(i + 1 < SA)
        def _():
            wait_idx(nu)

        @pl.when(jnp.logical_and(i + 1 < SA, i >= 3))
        def _():
            wait_write(nu)

        @pl.when(i + 1 < SA)
        def _():
            issue_gather(nu)

        wait_gather(u)
        issue_write(u, i)

        @pl.when(i + 3 < SA)
        def _():
            issue_idx(pu, i + 3)

    @pl.loop(0, SA // NBUF)
    def _(it):
        for u in range(NBUF):
            step_fn(u, it * NBUF + u)

    for i in range((SA // NBUF) * NBUF, SA):
        step_fn(i % NBUF, i)

    for u in range(NBUF):
        wait_write((SA - 4 + u) % NBUF)


def _sc_gather(q_tbl, kv_tbl, src, dst):
    buf = ([pltpu.VMEM((EA,), jnp.int32)] * NBUF
           + [pltpu.VMEM((EA,), jnp.int32)] * NBUF
           + [pltpu.VMEM((EA, FQK), jnp.float32)] * NBUF
           + [pltpu.VMEM((EA, FQK + FV), jnp.float32)] * NBUF
           + [pltpu.SemaphoreType.DMA] * (3 * NBUF))
    fn = pl.kernel(
        _sc_gather_body,
        compiler_params=_compiler_params(),
        out_type=[
            jax.ShapeDtypeStruct((E, FQK), jnp.float32),
            jax.ShapeDtypeStruct((E, FQK + FV), jnp.float32),
        ],
        mesh=_mesh(),
        scratch_types=buf,
    )
    return fn(q_tbl, kv_tbl, src, dst)


def _edge_body(src_ref, qs_ref, kvs_ref, wv_ref, dn_ref):
    blk = qs_ref.shape[0]
    qs = qs_ref[...]
    ks = kvs_ref[:, :FQK]
    vs = kvs_ref[:, FQK:]
    prod = qs * ks
    ch = lax.broadcasted_iota(jnp.int32, (FQK, H), 0) // FH
    hh = lax.broadcasted_iota(jnp.int32, (FQK, H), 1)
    sum16 = (ch == hh).astype(jnp.float32)
    hr = lax.broadcasted_iota(jnp.int32, (H, FV), 0)
    hc = lax.broadcasted_iota(jnp.int32, (H, FV), 1)
    expand = ((hc // FH) == hr).astype(jnp.float32)
    tile8 = ((hc & 15) == hr).astype(jnp.float32)

    aw = lax.dot_general(prod, sum16, (((1,), (0,)), ((), ())),
                         preferred_element_type=jnp.float32)
    w = jnp.exp(aw)
    wrep = lax.dot_general(w, expand, (((1,), (0,)), ((), ())),
                           preferred_element_type=jnp.float32)
    wv_ref[...] = wrep * vs
    wtile = lax.dot_general(w, tile8, (((1,), (0,)), ((), ())),
                            preferred_element_type=jnp.float32)
    grp = jnp.broadcast_to(src_ref[...] & 7, (blk, FV))
    lane16 = lax.broadcasted_iota(jnp.int32, (blk, FV), 1) // FH
    dn_ref[...] = jnp.where(lane16 == grp, wtile, 0.0)


def _edge_compute(src2, qs, kvs):
    blk = 2000
    return pl.pallas_call(
        _edge_body,
        grid=(E // blk,),
        in_specs=[
            pl.BlockSpec((blk, 1), lambda i: (i, 0)),
            pl.BlockSpec((blk, FQK), lambda i: (i, 0)),
            pl.BlockSpec((blk, FQK + FV), lambda i: (i, 0)),
        ],
        out_specs=[
            pl.BlockSpec((blk, FV), lambda i: (i, 0)),
            pl.BlockSpec((blk, 128), lambda i: (i, 0)),
        ],
        out_shape=[
            jax.ShapeDtypeStruct((E, FV), jnp.float32),
            jax.ShapeDtypeStruct((E, 128), jnp.float32),
        ],
    )(src2, qs, kvs)


def _sc_scatter_body(wv_hbm, dn_hbm, src_hbm, num_hbm, den_hbm,
                     s0, w0, n0, s1, w1, n1, st, didx, dt_, zv,
                     acc_n, acc_d, i0, i1, l0, l1, sz):
    cid = lax.axis_index("c")
    sid = lax.axis_index("s")
    tb = (cid * 16 + sid) * EPT
    zero16 = jnp.zeros((16,), jnp.float32)
    S = (s0, s1)
    Wb = (w0, w1)
    Nb = (n0, n1)
    SI = (i0, i1)
    SL = (l0, l1)

    @pl.loop(0, ZB)
    def _(i):
        for j in range(FV // 16):
            zv[i, pl.ds(16 * j, 16)] = zero16

    ztargets = [acc_n.at[pl.ds(sid * RPT + r * ZB, ZB)]
                for r in range(RPT // ZB)]
    ztargets += [acc_d.at[pl.ds(sid * DPT + r * ZB, ZB)]
                 for r in range(DPT // ZB)]
    for wave in range(0, len(ztargets), 15):
        hs = [pltpu.async_copy(zv, t, sz) for t in ztargets[wave:wave + 15]]
        for hh in hs:
            hh.wait()
    plsc.subcore_barrier()

    def issue_load(b, step):
        base = tb + step * EC
        pltpu.async_copy(src_hbm.at[pl.ds(base, EC)], S[b], SI[b])
        pltpu.async_copy(wv_hbm.at[pl.ds(base, EC)], Wb[b], SL[b])
        pltpu.async_copy(dn_hbm.at[pl.ds(base, EC)], Nb[b], SL[b])

    def wait_load(b):
        pltpu.make_async_copy(src_hbm.at[pl.ds(0, EC)], S[b], SI[b]).wait()
        pltpu.make_async_copy(wv_hbm.at[pl.ds(0, EC)], Wb[b], SL[b]).wait()
        pltpu.make_async_copy(dn_hbm.at[pl.ds(0, EC)], Nb[b], SL[b]).wait()

    issue_load(0, 0)
    issue_load(1, 1)

    @pl.loop(0, SC // 2)
    def _(it):
        for b in (0, 1):
            i = it * 2 + b
            wait_load(b)

            @pl.loop(0, EC // 16)
            def _(c):
                didx[pl.ds(c * 16, 16)] = lax.shift_right_logical(
                    S[b][pl.ds(c * 16, 16)], 3)

            pltpu.sync_copy(Wb[b], acc_n.at[S[b]], add=True)
            pltpu.sync_copy(Nb[b], acc_d.at[didx], add=True)

            @pl.when(i + 2 < SC)
            def _():
                issue_load(b, i + 2)

    base_t = tb + SC * EC
    pltpu.sync_copy(src_hbm.at[pl.ds(base_t, TC)], st)
    pltpu.sync_copy(wv_hbm.at[pl.ds(base_t, TC)], w0.at[pl.ds(0, TC)])
    pltpu.sync_copy(dn_hbm.at[pl.ds(base_t, TC)], n0.at[pl.ds(0, TC)])
    dt_[pl.ds(0, 16)] = lax.shift_right_logical(st[pl.ds(0, 16)], 3)
    pltpu.sync_copy(w0.at[pl.ds(0, TC)], acc_n.at[st], add=True)
    pltpu.sync_copy(n0.at[pl.ds(0, TC)], acc_d.at[dt_], add=True)

    plsc.subcore_barrier()
    for r in range(RPT // EC):
        pltpu.sync_copy(acc_n.at[pl.ds(sid * RPT + r * EC, EC)], w0)
        pltpu.sync_copy(w0, num_hbm.at[cid, pl.ds(sid * RPT + r * EC, EC)])
    for r in range(DPT // 16):
        pltpu.sync_copy(acc_d.at[pl.ds(sid * DPT + r * 16, 16)],
                        w0.at[pl.ds(0, 16)])
        pltpu.sync_copy(w0.at[pl.ds(0, 16)],
                        den_hbm.at[cid, pl.ds(sid * DPT + r * 16, 16)])


def _sc_scatter(wv, dn, src):
    dbuf = [
        pltpu.VMEM((EC,), jnp.int32),
        pltpu.VMEM((EC, FV), jnp.float32),
        pltpu.VMEM((EC, 128), jnp.float32),
    ]
    fn = pl.kernel(
        _sc_scatter_body,
        compiler_params=_compiler_params(),
        out_type=[
            jax.ShapeDtypeStruct((2, NP, FV), jnp.float32),
            jax.ShapeDtypeStruct((2, ND, 128), jnp.float32),
        ],
        mesh=_mesh(),
        scratch_types=dbuf + dbuf + [
            pltpu.VMEM((TC,), jnp.int32),
            pltpu.VMEM((EC,), jnp.int32),
            pltpu.VMEM((TC,), jnp.int32),
            pltpu.VMEM((ZB, 128), jnp.float32),
            pltpu.VMEM_SHARED((NP, FV), jnp.float32),
            pltpu.VMEM_SHARED((ND, 128), jnp.float32),
        ] + [pltpu.SemaphoreType.DMA] * 5,
    )
    return fn(wv, dn, src)


def _comb_body(num_ref, den_ref, o_ref):
    num = num_ref[0] + num_ref[1]
    den16 = den_ref[0] + den_ref[1]
    col = lax.broadcasted_iota(jnp.int32, (16, FV), 1) // FH
    row = lax.broadcasted_iota(jnp.int32, (16, FV), 0)
    ex = (col == row).astype(jnp.float32)
    rep = lax.dot_general(den16, ex, (((1,), (0,)), ((), ())),
                          preferred_element_type=jnp.float32)
    o_ref[...] = jnp.where(rep > 0, num / rep, 0.0)


def _combine(nd_num, nd_den16):
    blk = 1000
    return pl.pallas_call(
        _comb_body,
        grid=(N // blk,),
        in_specs=[
            pl.BlockSpec((2, blk, FV), lambda i: (0, i, 0)),
            pl.BlockSpec((2, blk, 16), lambda i: (0, i, 0)),
        ],
        out_specs=pl.BlockSpec((blk, FV), lambda i: (i, 0)),
        out_shape=jax.ShapeDtypeStruct((N, FV), jnp.float32),
    )(nd_num, nd_den16)


def kernel(x, batch, ei, W):
    del batch
    src = ei[0]
    dst = ei[1]
    q_tbl, kv_tbl = _project(x, W)
    qs, kvs = _sc_gather(q_tbl, kv_tbl, src, dst)
    wv, dn = _edge_compute(src.reshape(E, 1), qs, kvs)
    nd_num, nd_den = _sc_scatter(wv, dn, src)
    return _combine(nd_num, nd_den.reshape(2, NP, 16))


# speedup vs baseline: 1.3458x; 1.1443x over previous
import dataclasses
import functools

import jax
import jax.numpy as jnp
from jax import lax
from jax.experimental import pallas as pl
from jax.experimental.pallas import tpu as pltpu
from jax.experimental.pallas import tpu_sc as plsc

N = 10000
E = 320000
FIN = 128
FQK = 128
FV = 128
H = 8
FH = 16
NTILES = 32
EPT = E // NTILES
NP = 10240
ND = NP // 8
RPT = NP // 16
DPT = ND // 16
ZB = 16

EA = 80
SA = EPT // EA
NBUF = 4

EC = 64
SC = EPT // EC
TC = EPT - SC * EC


def _compiler_params():
    cp = pltpu.CompilerParams()
    if "needs_layout_passes" in pltpu.CompilerParams.__dataclass_fields__:
        cp = dataclasses.replace(cp, needs_layout_passes=False)
    return cp


def _mesh():
    return plsc.VectorSubcoreMesh(core_axis_name="c", subcore_axis_name="s")


def _pack_perm():
    f = lax.broadcasted_iota(jnp.int32, (FQK, FQK), 0)
    c = lax.broadcasted_iota(jnp.int32, (FQK, FQK), 1)
    return (f == 2 * (c % 64) + (c // 64)).astype(jnp.float32)


def _proj_body(x_ref, w_ref, q_ref, kv_ref):
    p = lax.dot_general(x_ref[...], w_ref[...], (((1,), (1,)), ((), ())),
                        preferred_element_type=jnp.float32)
    q = p[:, :FQK] * (FH ** -0.5)
    q_ref[...] = lax.dot_general(q, _pack_perm(), (((1,), (0,)), ((), ())),
                                 preferred_element_type=jnp.float32)
    kv_f = p[:, FQK:]
    cc = lax.broadcasted_iota(jnp.int32, (FQK + FV, (FQK + FV) // 2), 0)
    jj = lax.broadcasted_iota(jnp.int32, (FQK + FV, (FQK + FV) // 2), 1)
    sel_even = (cc == 2 * jj).astype(jnp.float32)
    sel_odd = (cc == 2 * jj + 1).astype(jnp.float32)
    ke = lax.dot_general(kv_f, sel_even, (((1,), (0,)), ((), ())),
                         preferred_element_type=jnp.float32)
    ko = lax.dot_general(kv_f, sel_odd, (((1,), (0,)), ((), ())),
                         preferred_element_type=jnp.float32)

    def b16(xf):
        u = lax.bitcast_convert_type(xf, jnp.int32)
        r = u + 0x7FFF + (lax.shift_right_logical(u, 16) & 1)
        return lax.shift_right_logical(r, 16)

    kv_ref[...] = b16(ke) | (b16(ko) << 16)


def _project(x, W):
    blk = 1000
    return pl.pallas_call(
        _proj_body,
        grid=(N // blk,),
        in_specs=[
            pl.BlockSpec((blk, FIN), lambda i: (i, 0)),
            pl.BlockSpec((2 * FQK + FV, FIN), lambda i: (0, 0)),
        ],
        out_specs=[
            pl.BlockSpec((blk, FQK), lambda i: (i, 0)),
            pl.BlockSpec((blk, (FQK + FV) // 2), lambda i: (i, 0)),
        ],
        out_shape=[
            jax.ShapeDtypeStruct((N, FQK), jnp.float32),
            jax.ShapeDtypeStruct((N, (FQK + FV) // 2), jnp.int32),
        ],
    )(x, W)


def _sc_gather_body(q_hbm, kv_hbm, src_hbm, dst_hbm, qs_hbm, kvs_hbm,
                    *scratch):
    S = scratch[0:NBUF]
    D = scratch[NBUF:2 * NBUF]
    Q = scratch[2 * NBUF:3 * NBUF]
    K = scratch[3 * NBUF:4 * NBUF]
    SI = scratch[4 * NBUF:5 * NBUF]
    SG = scratch[5 * NBUF:6 * NBUF]
    SW = scratch[6 * NBUF:7 * NBUF]
    cid = lax.axis_index("c")
    sid = lax.axis_index("s")
    tb = (cid * 16 + sid) * EPT

    def issue_idx(u, step):
        base = tb + step * EA
        pltpu.async_copy(src_hbm.at[pl.ds(base, EA)], S[u], SI[u])
        pltpu.async_copy(dst_hbm.at[pl.ds(base, EA)], D[u], SI[u])

    def wait_idx(u):
        pltpu.make_async_copy(src_hbm.at[pl.ds(0, EA)], S[u], SI[u]).wait()
        pltpu.make_async_copy(dst_hbm.at[pl.ds(0, EA)], D[u], SI[u]).wait()

    def issue_gather(u):
        pltpu.async_copy(q_hbm.at[S[u]], Q[u], SG[u])
        pltpu.async_copy(kv_hbm.at[D[u]], K[u], SG[u])

    def wait_gather(u):
        pltpu.make_async_copy(q_hbm.at[S[u]], Q[u], SG[u]).wait()
        pltpu.make_async_copy(kv_hbm.at[D[u]], K[u], SG[u]).wait()

    def issue_write(u, step):
        base = tb + step * EA
        pltpu.async_copy(Q[u], qs_hbm.at[pl.ds(base, EA)], SW[u])
        pltpu.async_copy(K[u], kvs_hbm.at[pl.ds(base, EA)], SW[u])

    def wait_write(u):
        pltpu.make_async_copy(Q[u], qs_hbm.at[pl.ds(0, EA)], SW[u]).wait()
        pltpu.make_async_copy(K[u], kvs_hbm.at[pl.ds(0, EA)], SW[u]).wait()

    issue_idx(0, 0)
    issue_idx(1, 1)
    issue_idx(2, 2)
    wait_idx(0)
    issue_gather(0)

    def step_fn(u, i):
        nu = (u + 1) % NBUF
        pu = (u + NBUF - 1) % NBUF

        @pl.when(i + 1 < SA)
        def _():
            wait_idx(nu)

        @pl.when(jnp.logical_and(i + 1 < SA, i >= 3))
        def _():
            wait_write(nu)

        @pl.when(i + 1 < SA)
        def _():
            issue_gather(nu)

        wait_gather(u)
        issue_write(u, i)

        @pl.when(i + 3 < SA)
        def _():
            issue_idx(pu, i + 3)

    @pl.loop(0, SA // NBUF)
    def _(it):
        for u in range(NBUF):
            step_fn(u, it * NBUF + u)

    for i in range((SA // NBUF) * NBUF, SA):
        step_fn(i % NBUF, i)

    for u in range(NBUF):
        wait_write((SA - 4 + u) % NBUF)


def _sc_gather(q_tbl, kv_i, src, dst):
    buf = ([pltpu.VMEM((EA,), jnp.int32)] * NBUF
           + [pltpu.VMEM((EA,), jnp.int32)] * NBUF
           + [pltpu.VMEM((EA, FQK), jnp.float32)] * NBUF
           + [pltpu.VMEM((EA, 128), jnp.int32)] * NBUF
           + [pltpu.SemaphoreType.DMA] * (3 * NBUF))
    fn = pl.kernel(
        _sc_gather_body,
        compiler_params=_compiler_params(),
        out_type=[
            jax.ShapeDtypeStruct((E, FQK), jnp.float32),
            jax.ShapeDtypeStruct((E, 128), jnp.int32),
        ],
        mesh=_mesh(),
        scratch_types=buf,
    )
    return fn(q_tbl, kv_i, src, dst)


def _edge_body(src_ref, qs_ref, kvs_ref, wv_ref, dn_ref):
    blk = qs_ref.shape[0]
    qp = qs_ref[...]
    kvi = kvs_ref[...]
    ki = kvi[:, :64]
    vi = kvi[:, 64:]
    himask = jnp.int32(-65536)

    def unlo(z):
        return lax.bitcast_convert_type(z << 16, jnp.float32)

    def unhi(z):
        return lax.bitcast_convert_type(z & himask, jnp.float32)

    prodh = qp[:, :64] * unlo(ki) + qp[:, 64:] * unhi(ki)
    vp = jnp.concatenate([unlo(vi), unhi(vi)], axis=1)
    ch = lax.broadcasted_iota(jnp.int32, (64, H), 0) // (FH // 2)
    hh = lax.broadcasted_iota(jnp.int32, (64, H), 1)
    sum8 = (ch == hh).astype(jnp.float32)
    hr = lax.broadcasted_iota(jnp.int32, (H, FV), 0)
    hc = lax.broadcasted_iota(jnp.int32, (H, FV), 1)
    expand_p = (((hc % 64) // (FH // 2)) == hr).astype(jnp.float32)
    tile8 = ((hc & 15) == hr).astype(jnp.float32)

    aw = lax.dot_general(prodh, sum8, (((1,), (0,)), ((), ())),
                         preferred_element_type=jnp.float32)
    w = jnp.exp(aw)
    wrep = lax.dot_general(w, expand_p, (((1,), (0,)), ((), ())),
                           preferred_element_type=jnp.float32)
    wv_ref[...] = wrep * vp
    wtile = lax.dot_general(w, tile8, (((1,), (0,)), ((), ())),
                            preferred_element_type=jnp.float32)
    grp = jnp.broadcast_to(src_ref[...] & 7, (blk, FV))
    lane16 = lax.broadcasted_iota(jnp.int32, (blk, FV), 1) // FH
    dn_ref[...] = jnp.where(lane16 == grp, wtile, 0.0)


def _edge_compute(src2, qs, kvs):
    blk = 2000
    return pl.pallas_call(
        _edge_body,
        grid=(E // blk,),
        in_specs=[
            pl.BlockSpec((blk, 1), lambda i: (i, 0)),
            pl.BlockSpec((blk, FQK), lambda i: (i, 0)),
            pl.BlockSpec((blk, 128), lambda i: (i, 0)),
        ],
        out_specs=[
            pl.BlockSpec((blk, FV), lambda i: (i, 0)),
            pl.BlockSpec((blk, 128), lambda i: (i, 0)),
        ],
        out_shape=[
            jax.ShapeDtypeStruct((E, FV), jnp.float32),
            jax.ShapeDtypeStruct((E, 128), jnp.float32),
        ],
    )(src2, qs, kvs)


def _sc_scatter_body(wv_hbm, dn_hbm, src_hbm, num_hbm, den_hbm,
                     s0, w0, n0, s1, w1, n1, st, didx, dt_, zv,
                     acc_n, acc_d, i0, i1, l0, l1, sz):
    cid = lax.axis_index("c")
    sid = lax.axis_index("s")
    tb = (cid * 16 + sid) * EPT
    zero16 = jnp.zeros((16,), jnp.float32)
    S = (s0, s1)
    Wb = (w0, w1)
    Nb = (n0, n1)
    SI = (i0, i1)
    SL = (l0, l1)

    @pl.loop(0, ZB)
    def _(i):
        for j in range(FV // 16):
            zv[i, pl.ds(16 * j, 16)] = zero16

    ztargets = [acc_n.at[pl.ds(sid * RPT + r * ZB, ZB)]
                for r in range(RPT // ZB)]
    ztargets += [acc_d.at[pl.ds(sid * DPT + r * ZB, ZB)]
                 for r in range(DPT // ZB)]
    for wave in range(0, len(ztargets), 15):
        hs = [pltpu.async_copy(zv, t, sz) for t in ztargets[wave:wave + 15]]
        for hh in hs:
            hh.wait()
    plsc.subcore_barrier()

    def issue_load(b, step):
        base = tb + step * EC
        pltpu.async_copy(src_hbm.at[pl.ds(base, EC)], S[b], SI[b])
        pltpu.async_copy(wv_hbm.at[pl.ds(base, EC)], Wb[b], SL[b])
        pltpu.async_copy(dn_hbm.at[pl.ds(base, EC)], Nb[b], SL[b])

    def wait_load(b):
        pltpu.make_async_copy(src_hbm.at[pl.ds(0, EC)], S[b], SI[b]).wait()
        pltpu.make_async_copy(wv_hbm.at[pl.ds(0, EC)], Wb[b], SL[b]).wait()
        pltpu.make_async_copy(dn_hbm.at[pl.ds(0, EC)], Nb[b], SL[b]).wait()

    issue_load(0, 0)
    issue_load(1, 1)

    @pl.loop(0, SC // 2)
    def _(it):
        for b in (0, 1):
            i = it * 2 + b
            wait_load(b)

            @pl.loop(0, EC // 16)
            def _(c):
                didx[pl.ds(c * 16, 16)] = lax.shift_right_logical(
                    S[b][pl.ds(c * 16, 16)], 3)

            pltpu.sync_copy(Wb[b], acc_n.at[S[b]], add=True)
            pltpu.sync_copy(Nb[b], acc_d.at[didx], add=True)

            @pl.when(i + 2 < SC)
            def _():
                issue_load(b, i + 2)

    base_t = tb + SC * EC
    pltpu.sync_copy(src_hbm.at[pl.ds(base_t, TC)], st)
    pltpu.sync_copy(wv_hbm.at[pl.ds(base_t, TC)], w0.at[pl.ds(0, TC)])
    pltpu.sync_copy(dn_hbm.at[pl.ds(base_t, TC)], n0.at[pl.ds(0, TC)])
    dt_[pl.ds(0, 16)] = lax.shift_right_logical(st[pl.ds(0, 16)], 3)
    pltpu.sync_copy(w0.at[pl.ds(0, TC)], acc_n.at[st], add=True)
    pltpu.sync_copy(n0.at[pl.ds(0, TC)], acc_d.at[dt_], add=True)

    plsc.subcore_barrier()
    for r in range(RPT // EC):
        pltpu.sync_copy(acc_n.at[pl.ds(sid * RPT + r * EC, EC)], w0)
        pltpu.sync_copy(w0, num_hbm.at[cid, pl.ds(sid * RPT + r * EC, EC)])
    for r in range(DPT // 16):
        pltpu.sync_copy(acc_d.at[pl.ds(sid * DPT + r * 16, 16)],
                        w0.at[pl.ds(0, 16)])
        pltpu.sync_copy(w0.at[pl.ds(0, 16)],
                        den_hbm.at[cid, pl.ds(sid * DPT + r * 16, 16)])


def _sc_scatter(wv, dn, src):
    dbuf = [
        pltpu.VMEM((EC,), jnp.int32),
        pltpu.VMEM((EC, FV), jnp.float32),
        pltpu.VMEM((EC, 128), jnp.float32),
    ]
    fn = pl.kernel(
        _sc_scatter_body,
        compiler_params=_compiler_params(),
        out_type=[
            jax.ShapeDtypeStruct((2, NP, FV), jnp.float32),
            jax.ShapeDtypeStruct((2, ND, 128), jnp.float32),
        ],
        mesh=_mesh(),
        scratch_types=dbuf + dbuf + [
            pltpu.VMEM((TC,), jnp.int32),
            pltpu.VMEM((EC,), jnp.int32),
            pltpu.VMEM((TC,), jnp.int32),
            pltpu.VMEM((ZB, 128), jnp.float32),
            pltpu.VMEM_SHARED((NP, FV), jnp.float32),
            pltpu.VMEM_SHARED((ND, 128), jnp.float32),
        ] + [pltpu.SemaphoreType.DMA] * 5,
    )
    return fn(wv, dn, src)


def _comb_body(num_ref, den_ref, o_ref):
    num = num_ref[0] + num_ref[1]
    den16 = den_ref[0] + den_ref[1]
    col = (lax.broadcasted_iota(jnp.int32, (16, FV), 1) % 64) // (FH // 2)
    row = lax.broadcasted_iota(jnp.int32, (16, FV), 0)
    ex = (col == row).astype(jnp.float32)
    rep = lax.dot_general(den16, ex, (((1,), (0,)), ((), ())),
                          preferred_element_type=jnp.float32)
    outp = jnp.where(rep > 0, num / rep, 0.0)
    o_ref[...] = lax.dot_general(outp, _pack_perm(), (((1,), (1,)), ((), ())),
                                 preferred_element_type=jnp.float32)


def _combine(nd_num, nd_den16):
    blk = 1000
    return pl.pallas_call(
        _comb_body,
        grid=(N // blk,),
        in_specs=[
            pl.BlockSpec((2, blk, FV), lambda i: (0, i, 0)),
            pl.BlockSpec((2, blk, 16), lambda i: (0, i, 0)),
        ],
        out_specs=pl.BlockSpec((blk, FV), lambda i: (i, 0)),
        out_shape=jax.ShapeDtypeStruct((N, FV), jnp.float32),
    )(nd_num, nd_den16)


def kernel(x, batch, ei, W):
    del batch
    src = ei[0]
    dst = ei[1]
    q_tbl, kv_i = _project(x, W)
    qs, kvs = _sc_gather(q_tbl, kv_i, src, dst)
    wv, dn = _edge_compute(src.reshape(E, 1), qs, kvs)
    nd_num, nd_den = _sc_scatter(wv, dn, src)
    return _combine(nd_num, nd_den.reshape(2, NP, 16))


# speedup vs baseline: 1.4077x; 1.0459x over previous
import dataclasses
import functools

import jax
import jax.numpy as jnp
from jax import lax
from jax.experimental import pallas as pl
from jax.experimental.pallas import tpu as pltpu
from jax.experimental.pallas import tpu_sc as plsc

N = 10000
E = 320000
FIN = 128
FQK = 128
FV = 128
H = 8
FH = 16
NTILES = 32
EPT = E // NTILES
NP = 10240
ND = NP // 8
RPT = NP // 16
DPT = ND // 16
ZB = 16

EA = 80
SA = EPT // EA
NBUF = 4

EC = 64
SC = EPT // EC
TC = EPT - SC * EC


def _compiler_params():
    cp = pltpu.CompilerParams()
    if "needs_layout_passes" in pltpu.CompilerParams.__dataclass_fields__:
        cp = dataclasses.replace(cp, needs_layout_passes=False)
    return cp


def _mesh():
    return plsc.VectorSubcoreMesh(core_axis_name="c", subcore_axis_name="s")


def _pack_perm():
    f = lax.broadcasted_iota(jnp.int32, (FQK, FQK), 0)
    c = lax.broadcasted_iota(jnp.int32, (FQK, FQK), 1)
    return (f == 2 * (c % 64) + (c // 64)).astype(jnp.float32)


def _proj_body(x_ref, w_ref, q_ref, kv_ref):
    p = lax.dot_general(x_ref[...], w_ref[...], (((1,), (1,)), ((), ())),
                        preferred_element_type=jnp.float32)
    q = p[:, :FQK] * (FH ** -0.5)
    q_ref[...] = lax.dot_general(q, _pack_perm(), (((1,), (0,)), ((), ())),
                                 preferred_element_type=jnp.float32)
    kv_f = p[:, FQK:]
    cc = lax.broadcasted_iota(jnp.int32, (FQK + FV, (FQK + FV) // 2), 0)
    jj = lax.broadcasted_iota(jnp.int32, (FQK + FV, (FQK + FV) // 2), 1)
    sel_even = (cc == 2 * jj).astype(jnp.float32)
    sel_odd = (cc == 2 * jj + 1).astype(jnp.float32)
    ke = lax.dot_general(kv_f, sel_even, (((1,), (0,)), ((), ())),
                         preferred_element_type=jnp.float32)
    ko = lax.dot_general(kv_f, sel_odd, (((1,), (0,)), ((), ())),
                         preferred_element_type=jnp.float32)

    def b16(xf):
        u = lax.bitcast_convert_type(xf, jnp.int32)
        r = u + 0x7FFF + (lax.shift_right_logical(u, 16) & 1)
        return lax.shift_right_logical(r, 16)

    kv_ref[...] = b16(ke) | (b16(ko) << 16)


def _project(x, W):
    blk = 1000
    return pl.pallas_call(
        _proj_body,
        grid=(N // blk,),
        in_specs=[
            pl.BlockSpec((blk, FIN), lambda i: (i, 0)),
            pl.BlockSpec((2 * FQK + FV, FIN), lambda i: (0, 0)),
        ],
        out_specs=[
            pl.BlockSpec((blk, FQK), lambda i: (i, 0)),
            pl.BlockSpec((blk, (FQK + FV) // 2), lambda i: (i, 0)),
        ],
        out_shape=[
            jax.ShapeDtypeStruct((N, FQK), jnp.float32),
            jax.ShapeDtypeStruct((N, (FQK + FV) // 2), jnp.int32),
        ],
    )(x, W)


def _sc_gather_body(q_hbm, kv_hbm, src_hbm, dst_hbm, qs_hbm, kvs_hbm,
                    *scratch):
    S = scratch[0:NBUF]
    D = scratch[NBUF:2 * NBUF]
    Q = scratch[2 * NBUF:3 * NBUF]
    K = scratch[3 * NBUF:4 * NBUF]
    SI = scratch[4 * NBUF:5 * NBUF]
    SG = scratch[5 * NBUF:6 * NBUF]
    SW = scratch[6 * NBUF:7 * NBUF]
    cid = lax.axis_index("c")
    sid = lax.axis_index("s")
    tb = (cid * 16 + sid) * EPT

    def issue_idx(u, step):
        base = tb + step * EA
        pltpu.async_copy(src_hbm.at[pl.ds(base, EA)], S[u], SI[u])
        pltpu.async_copy(dst_hbm.at[pl.ds(base, EA)], D[u], SI[u])

    def wait_idx(u):
        pltpu.make_async_copy(src_hbm.at[pl.ds(0, EA)], S[u], SI[u]).wait()
        pltpu.make_async_copy(dst_hbm.at[pl.ds(0, EA)], D[u], SI[u]).wait()

    def issue_gather(u):
        pltpu.async_copy(q_hbm.at[S[u]], Q[u], SG[u])
        pltpu.async_copy(kv_hbm.at[D[u]], K[u], SG[u])

    def wait_gather(u):
        pltpu.make_async_copy(q_hbm.at[S[u]], Q[u], SG[u]).wait()
        pltpu.make_async_copy(kv_hbm.at[D[u]], K[u], SG[u]).wait()

    def issue_write(u, step):
        base = tb + step * EA
        pltpu.async_copy(Q[u], qs_hbm.at[pl.ds(base, EA)], SW[u])
        pltpu.async_copy(K[u], kvs_hbm.at[pl.ds(base, EA)], SW[u])

    def wait_write(u):
        pltpu.make_async_copy(Q[u], qs_hbm.at[pl.ds(0, EA)], SW[u]).wait()
        pltpu.make_async_copy(K[u], kvs_hbm.at[pl.ds(0, EA)], SW[u]).wait()

    issue_idx(0, 0)
    issue_idx(1, 1)
    issue_idx(2, 2)
    wait_idx(0)
    issue_gather(0)

    def step_fn(u, i):
        nu = (u + 1) % NBUF
        pu = (u + NBUF - 1) % NBUF

        @pl.when(i + 1 < SA)
        def _():
            wait_idx(nu)

        @pl.when(jnp.logical_and(i + 1 < SA, i >= 3))
        def _():
            wait_write(nu)

        @pl.when(i + 1 < SA)
        def _():
            issue_gather(nu)

        wait_gather(u)
        issue_write(u, i)

        @pl.when(i + 3 < SA)
        def _():
            issue_idx(pu, i + 3)

    @pl.loop(0, SA // NBUF)
    def _(it):
        for u in range(NBUF):
            step_fn(u, it * NBUF + u)

    for i in range((SA // NBUF) * NBUF, SA):
        step_fn(i % NBUF, i)

    for u in range(NBUF):
        wait_write((SA - 4 + u) % NBUF)


def _sc_gather(q_tbl, kv_i, src, dst):
    buf = ([pltpu.VMEM((EA,), jnp.int32)] * NBUF
           + [pltpu.VMEM((EA,), jnp.int32)] * NBUF
           + [pltpu.VMEM((EA, FQK), jnp.float32)] * NBUF
           + [pltpu.VMEM((EA, 128), jnp.int32)] * NBUF
           + [pltpu.SemaphoreType.DMA] * (3 * NBUF))
    fn = pl.kernel(
        _sc_gather_body,
        compiler_params=_compiler_params(),
        out_type=[
            jax.ShapeDtypeStruct((E, FQK), jnp.float32),
            jax.ShapeDtypeStruct((E, 128), jnp.int32),
        ],
        mesh=_mesh(),
        scratch_types=buf,
    )
    return fn(q_tbl, kv_i, src, dst)


def _edge_body(src_ref, qs_ref, kvs_ref, wv_ref, dn_ref):
    blk = qs_ref.shape[0]
    qp = qs_ref[...]
    kvi = kvs_ref[...]
    ki = kvi[:, :64]
    vi = kvi[:, 64:]
    himask = jnp.int32(-65536)

    def unlo(z):
        return lax.bitcast_convert_type(z << 16, jnp.float32)

    def unhi(z):
        return lax.bitcast_convert_type(z & himask, jnp.float32)

    prodh = qp[:, :64] * unlo(ki) + qp[:, 64:] * unhi(ki)
    vp = jnp.concatenate([unlo(vi), unhi(vi)], axis=1)
    ch = lax.broadcasted_iota(jnp.int32, (64, H), 0) // (FH // 2)
    hh = lax.broadcasted_iota(jnp.int32, (64, H), 1)
    sum8 = (ch == hh).astype(jnp.float32)
    hr = lax.broadcasted_iota(jnp.int32, (H, FV), 0)
    hc = lax.broadcasted_iota(jnp.int32, (H, FV), 1)
    expand_p = (((hc % 64) // (FH // 2)) == hr).astype(jnp.float32)
    tile8 = ((hc & 15) == hr).astype(jnp.float32)

    aw = lax.dot_general(prodh, sum8, (((1,), (0,)), ((), ())),
                         preferred_element_type=jnp.float32)
    w = jnp.exp(aw)
    wrep = lax.dot_general(w, expand_p, (((1,), (0,)), ((), ())),
                           preferred_element_type=jnp.float32)
    wv_ref[...] = wrep * vp
    wtile = lax.dot_general(w, tile8, (((1,), (0,)), ((), ())),
                            preferred_element_type=jnp.float32)
    grp = jnp.broadcast_to(src_ref[...] & 7, (blk, FV))
    lane16 = lax.broadcasted_iota(jnp.int32, (blk, FV), 1) // FH
    dn_ref[...] = jnp.where(lane16 == grp, wtile, 0.0)


def _edge_compute(src2, qs, kvs):
    blk = 4000
    return pl.pallas_call(
        _edge_body,
        grid=(E // blk,),
        in_specs=[
            pl.BlockSpec((blk, 1), lambda i: (i, 0)),
            pl.BlockSpec((blk, FQK), lambda i: (i, 0)),
            pl.BlockSpec((blk, 128), lambda i: (i, 0)),
        ],
        out_specs=[
            pl.BlockSpec((blk, FV), lambda i: (i, 0)),
            pl.BlockSpec((blk, 128), lambda i: (i, 0)),
        ],
        out_shape=[
            jax.ShapeDtypeStruct((E, FV), jnp.float32),
            jax.ShapeDtypeStruct((E, 128), jnp.float32),
        ],
    )(src2, qs, kvs)


def _sc_scatter_body(wv_hbm, dn_hbm, src_hbm, num_hbm, den_hbm,
                     s0, w0, n0, s1, w1, n1, st, didx, dt_, zv,
                     acc_n, acc_d, i0, i1, l0, l1, sz):
    cid = lax.axis_index("c")
    sid = lax.axis_index("s")
    tb = (cid * 16 + sid) * EPT
    zero16 = jnp.zeros((16,), jnp.float32)
    S = (s0, s1)
    Wb = (w0, w1)
    Nb = (n0, n1)
    SI = (i0, i1)
    SL = (l0, l1)

    @pl.loop(0, ZB)
    def _(i):
        for j in range(FV // 16):
            zv[i, pl.ds(16 * j, 16)] = zero16

    ztargets = [acc_n.at[pl.ds(sid * RPT + r * ZB, ZB)]
                for r in range(RPT // ZB)]
    ztargets += [acc_d.at[pl.ds(sid * DPT + r * ZB, ZB)]
                 for r in range(DPT // ZB)]
    for wave in range(0, len(ztargets), 15):
        hs = [pltpu.async_copy(zv, t, sz) for t in ztargets[wave:wave + 15]]
        for hh in hs:
            hh.wait()
    plsc.subcore_barrier()

    def issue_load(b, step):
        base = tb + step * EC
        pltpu.async_copy(src_hbm.at[pl.ds(base, EC)], S[b], SI[b])
        pltpu.async_copy(wv_hbm.at[pl.ds(base, EC)], Wb[b], SL[b])
        pltpu.async_copy(dn_hbm.at[pl.ds(base, EC)], Nb[b], SL[b])

    def wait_load(b):
        pltpu.make_async_copy(src_hbm.at[pl.ds(0, EC)], S[b], SI[b]).wait()
        pltpu.make_async_copy(wv_hbm.at[pl.ds(0, EC)], Wb[b], SL[b]).wait()
        pltpu.make_async_copy(dn_hbm.at[pl.ds(0, EC)], Nb[b], SL[b]).wait()

    issue_load(0, 0)
    issue_load(1, 1)

    @pl.loop(0, SC // 2)
    def _(it):
        for b in (0, 1):
            i = it * 2 + b
            wait_load(b)

            @pl.loop(0, EC // 16)
            def _(c):
                didx[pl.ds(c * 16, 16)] = lax.shift_right_logical(
                    S[b][pl.ds(c * 16, 16)], 3)

            pltpu.sync_copy(Wb[b], acc_n.at[S[b]], add=True)
            pltpu.sync_copy(Nb[b], acc_d.at[didx], add=True)

            @pl.when(i + 2 < SC)
            def _():
                issue_load(b, i + 2)

    base_t = tb + SC * EC
    pltpu.sync_copy(src_hbm.at[pl.ds(base_t, TC)], st)
    pltpu.sync_copy(wv_hbm.at[pl.ds(base_t, TC)], w0.at[pl.ds(0, TC)])
    pltpu.sync_copy(dn_hbm.at[pl.ds(base_t, TC)], n0.at[pl.ds(0, TC)])
    dt_[pl.ds(0, 16)] = lax.shift_right_logical(st[pl.ds(0, 16)], 3)
    pltpu.sync_copy(w0.at[pl.ds(0, TC)], acc_n.at[st], add=True)
    pltpu.sync_copy(n0.at[pl.ds(0, TC)], acc_d.at[dt_], add=True)

    plsc.subcore_barrier()
    for r in range(RPT // EC):
        pltpu.sync_copy(acc_n.at[pl.ds(sid * RPT + r * EC, EC)], w0)
        pltpu.sync_copy(w0, num_hbm.at[cid, pl.ds(sid * RPT + r * EC, EC)])
    for r in range(DPT // 16):
        pltpu.sync_copy(acc_d.at[pl.ds(sid * DPT + r * 16, 16)],
                        w0.at[pl.ds(0, 16)])
        pltpu.sync_copy(w0.at[pl.ds(0, 16)],
                        den_hbm.at[cid, pl.ds(sid * DPT + r * 16, 16)])


def _sc_scatter(wv, dn, src):
    dbuf = [
        pltpu.VMEM((EC,), jnp.int32),
        pltpu.VMEM((EC, FV), jnp.float32),
        pltpu.VMEM((EC, 128), jnp.float32),
    ]
    fn = pl.kernel(
        _sc_scatter_body,
        compiler_params=_compiler_params(),
        out_type=[
            jax.ShapeDtypeStruct((2, NP, FV), jnp.float32),
            jax.ShapeDtypeStruct((2, ND, 128), jnp.float32),
        ],
        mesh=_mesh(),
        scratch_types=dbuf + dbuf + [
            pltpu.VMEM((TC,), jnp.int32),
            pltpu.VMEM((EC,), jnp.int32),
            pltpu.VMEM((TC,), jnp.int32),
            pltpu.VMEM((ZB, 128), jnp.float32),
            pltpu.VMEM_SHARED((NP, FV), jnp.float32),
            pltpu.VMEM_SHARED((ND, 128), jnp.float32),
        ] + [pltpu.SemaphoreType.DMA] * 5,
    )
    return fn(wv, dn, src)


def _comb_body(num_ref, den_ref, o_ref):
    num = num_ref[0] + num_ref[1]
    den16 = den_ref[0] + den_ref[1]
    col = (lax.broadcasted_iota(jnp.int32, (16, FV), 1) % 64) // (FH // 2)
    row = lax.broadcasted_iota(jnp.int32, (16, FV), 0)
    ex = (col == row).astype(jnp.float32)
    rep = lax.dot_general(den16, ex, (((1,), (0,)), ((), ())),
                          preferred_element_type=jnp.float32)
    outp = jnp.where(rep > 0, num / rep, 0.0)
    o_ref[...] = lax.dot_general(outp, _pack_perm(), (((1,), (1,)), ((), ())),
                                 preferred_element_type=jnp.float32)


def _combine(nd_num, nd_den16):
    blk = 1000
    return pl.pallas_call(
        _comb_body,
        grid=(N // blk,),
        in_specs=[
            pl.BlockSpec((2, blk, FV), lambda i: (0, i, 0)),
            pl.BlockSpec((2, blk, 16), lambda i: (0, i, 0)),
        ],
        out_specs=pl.BlockSpec((blk, FV), lambda i: (i, 0)),
        out_shape=jax.ShapeDtypeStruct((N, FV), jnp.float32),
    )(nd_num, nd_den16)


def kernel(x, batch, ei, W):
    del batch
    src = ei[0]
    dst = ei[1]
    q_tbl, kv_i = _project(x, W)
    qs, kvs = _sc_gather(q_tbl, kv_i, src, dst)
    wv, dn = _edge_compute(src.reshape(E, 1), qs, kvs)
    nd_num, nd_den = _sc_scatter(wv, dn, src)
    return _combine(nd_num, nd_den.reshape(2, NP, 16))
